# Initial kernel scaffold; baseline (speedup 1.0000x reference)
import jax
import jax.numpy as jnp
from jax.experimental import pallas as pl


def kernel(layer_values, ordinals, weights):
    raise NotImplementedError("write your pallas kernel here")


# speedup vs baseline: 2.5863x; 2.5863x over previous
import functools

import jax
import jax.numpy as jnp
from jax import lax
from jax.experimental import pallas as pl
from jax.experimental.pallas import tpu as pltpu
from jax.experimental.pallas import tpu_sc as plsc

N_NODES = 10000
INPUTS_DIM = 32
D = 128
N_EDGES = N_NODES * INPUTS_DIM
B = N_EDGES // INPUTS_DIM

_INFO = plsc.get_sparse_core_info()
NC = _INFO.num_cores
NS = _INFO.num_subcores
NW = NC * NS
L = _INFO.num_lanes

CHUNK_ROWS = 8
CHUNK_IDX = CHUNK_ROWS * INPUTS_DIM
IDX_ROWS = CHUNK_IDX // 128
N_CHUNKS = B // CHUNK_ROWS
BASE_CHUNKS = N_CHUNKS // NW
EXTRA = N_CHUNKS - BASE_CHUNKS * NW


def _tanh(x):
    e = jnp.exp(x * 2.0)
    return 1.0 - 2.0 / (e + 1.0)


def _body(table_hbm, ord_hbm, w_hbm, out_hbm, idx_v, rows_v, w_v, out_v, sem):
    cid = lax.axis_index("c")
    sid = lax.axis_index("s")
    wid = sid * NC + cid

    pltpu.sync_copy(w_hbm, w_v)

    n_chunks = BASE_CHUNKS + jnp.where(wid < EXTRA, 1, 0)

    def chunk_body(t, carry):
        c = wid + t * NW
        pltpu.sync_copy(ord_hbm.at[pl.ds(c * IDX_ROWS, IDX_ROWS)], idx_v)
        cps = [
            pltpu.async_copy(
                table_hbm.at[idx_v.at[j]],
                rows_v.at[pl.ds(j * 128, 128)],
                sem,
            )
            for j in range(IDX_ROWS)
        ]
        for cp in cps:
            cp.wait()
        for d8 in range(D // L):
            dsl = pl.ds(d8 * L, L)
            wvs = [w_v[i, dsl] for i in range(INPUTS_DIM)]

            def rbody(r, rc):
                base = r * INPUTS_DIM
                acc = wvs[0] * rows_v[base, dsl]
                for i in range(1, INPUTS_DIM):
                    acc = acc + wvs[i] * rows_v[base + i, dsl]
                out_v[r, dsl] = _tanh(acc)
                return rc

            lax.fori_loop(0, CHUNK_ROWS, rbody, 0)
        pltpu.sync_copy(out_v, out_hbm.at[pl.ds(c * CHUNK_ROWS, CHUNK_ROWS)])
        return carry

    lax.fori_loop(0, n_chunks, chunk_body, 0)


@jax.jit
def _run(table, ords2d, weights):
    mesh = plsc.VectorSubcoreMesh(core_axis_name="c", subcore_axis_name="s")
    f = functools.partial(
        pl.kernel,
        mesh=mesh,
        out_type=jax.ShapeDtypeStruct((B, D), jnp.float32),
        scratch_types=[
            pltpu.VMEM((IDX_ROWS, 128), jnp.int32),
            pltpu.VMEM((CHUNK_IDX, D), jnp.float32),
            pltpu.VMEM((INPUTS_DIM, D), jnp.float32),
            pltpu.VMEM((CHUNK_ROWS, D), jnp.float32),
            pltpu.SemaphoreType.DMA,
        ],
    )(_body)
    return f(table, ords2d, weights)


def kernel(layer_values, ordinals, weights):
    table = layer_values.reshape(N_NODES, D)
    ords2d = ordinals.astype(jnp.int32).reshape(N_EDGES // 128, 128)
    out = _run(table, ords2d, weights)
    return out.reshape(B, D, 1)


# speedup vs baseline: 3.8120x; 1.4739x over previous
import functools

import jax
import jax.numpy as jnp
from jax import lax
from jax.experimental import pallas as pl
from jax.experimental.pallas import tpu as pltpu
from jax.experimental.pallas import tpu_sc as plsc

N_NODES = 10000
INPUTS_DIM = 32
D = 128
N_EDGES = N_NODES * INPUTS_DIM
B = N_EDGES // INPUTS_DIM

_INFO = plsc.get_sparse_core_info()
NC = _INFO.num_cores
NS = _INFO.num_subcores
NW = NC * NS
L = _INFO.num_lanes

CHUNK_ROWS = 8
CHUNK_IDX = CHUNK_ROWS * INPUTS_DIM
IDX_ROWS = CHUNK_IDX // 128
N_CHUNKS = B // CHUNK_ROWS
ITERS = -(-N_CHUNKS // NW)


def _tanh(x):
    e = jnp.exp(x * 2.0)
    return 1.0 - 2.0 / (e + 1.0)


def _body(table_hbm, ord_hbm, w_hbm, out_hbm,
          idx0, idx1, rows0, rows1, w_v, out0, out1,
          sem_i0, sem_i1, sem_g0, sem_g1, sem_o0, sem_o1):
    cid = lax.axis_index("c")
    sid = lax.axis_index("s")
    wid = sid * NC + cid

    idx = (idx0, idx1)
    rows = (rows0, rows1)
    outs = (out0, out1)
    sem_i = (sem_i0, sem_i1)
    sem_g = (sem_g0, sem_g1)
    sem_o = (sem_o0, sem_o1)

    pltpu.sync_copy(w_hbm, w_v)

    def chunk_id(t):
        return jnp.minimum(wid + t * NW, N_CHUNKS - 1)

    def ord_slice(c):
        return ord_hbm.at[pl.ds(c * IDX_ROWS, IDX_ROWS)]

    def issue_gathers(b):
        for j in range(IDX_ROWS):
            pltpu.async_copy(
                table_hbm.at[idx[b].at[j]],
                rows[b].at[pl.ds(j * 128, 128)],
                sem_g[b],
            )

    def wait_gathers(b):
        for j in range(IDX_ROWS):
            pltpu.make_async_copy(
                table_hbm.at[idx[b].at[j]],
                rows[b].at[pl.ds(j * 128, 128)],
                sem_g[b],
            ).wait()

    pltpu.sync_copy(ord_slice(chunk_id(0)), idx[0])
    issue_gathers(0)
    pltpu.async_copy(ord_slice(chunk_id(1)), idx[1], sem_i[1])

    def iter_body(p, carry):
        for b in range(2):
            t = p * 2 + b
            b1 = 1 - b
            wait_gathers(b)
            pltpu.async_copy(ord_slice(chunk_id(t + 2)), idx[b], sem_i[b])
            pltpu.make_async_copy(
                ord_slice(chunk_id(t + 1)), idx[b1], sem_i[b1]
            ).wait()
            issue_gathers(b1)
            @pl.when(t >= 2)
            def _():
                pltpu.make_async_copy(
                    outs[b],
                    out_hbm.at[pl.ds(chunk_id(t - 2) * CHUNK_ROWS, CHUNK_ROWS)],
                    sem_o[b],
                ).wait()
            c = chunk_id(t)
            for d8 in range(D // L):
                dsl = pl.ds(d8 * L, L)
                wvs = [w_v[i, dsl] for i in range(INPUTS_DIM)]

                def rbody(r, rc):
                    base = r * INPUTS_DIM
                    acc = wvs[0] * rows[b][base, dsl]
                    for i in range(1, INPUTS_DIM):
                        acc = acc + wvs[i] * rows[b][base + i, dsl]
                    outs[b][r, dsl] = _tanh(acc)
                    return rc

                lax.fori_loop(0, CHUNK_ROWS, rbody, 0)
            pltpu.async_copy(
                outs[b], out_hbm.at[pl.ds(c * CHUNK_ROWS, CHUNK_ROWS)], sem_o[b]
            )
        return carry

    lax.fori_loop(0, ITERS // 2, iter_body, 0)

    t_last = ITERS - 1
    pltpu.make_async_copy(ord_slice(chunk_id(t_last + 2)), idx[1], sem_i[1]).wait()
    wait_gathers(0)
    for b in range(2):
        t = t_last - 1 + b
        pltpu.make_async_copy(
            outs[b],
            out_hbm.at[pl.ds(chunk_id(t) * CHUNK_ROWS, CHUNK_ROWS)],
            sem_o[b],
        ).wait()


@jax.jit
def _run(table, ords2d, weights):
    mesh = plsc.VectorSubcoreMesh(core_axis_name="c", subcore_axis_name="s")
    f = functools.partial(
        pl.kernel,
        mesh=mesh,
        out_type=jax.ShapeDtypeStruct((B, D), jnp.float32),
        scratch_types=[
            pltpu.VMEM((IDX_ROWS, 128), jnp.int32),
            pltpu.VMEM((IDX_ROWS, 128), jnp.int32),
            pltpu.VMEM((CHUNK_IDX, D), jnp.float32),
            pltpu.VMEM((CHUNK_IDX, D), jnp.float32),
            pltpu.VMEM((INPUTS_DIM, D), jnp.float32),
            pltpu.VMEM((CHUNK_ROWS, D), jnp.float32),
            pltpu.VMEM((CHUNK_ROWS, D), jnp.float32),
            pltpu.SemaphoreType.DMA,
            pltpu.SemaphoreType.DMA,
            pltpu.SemaphoreType.DMA,
            pltpu.SemaphoreType.DMA,
            pltpu.SemaphoreType.DMA,
            pltpu.SemaphoreType.DMA,
        ],
    )(_body)
    return f(table, ords2d, weights)


def kernel(layer_values, ordinals, weights):
    table = layer_values.reshape(N_NODES, D)
    ords2d = ordinals.astype(jnp.int32).reshape(N_EDGES // 128, 128)
    out = _run(table, ords2d, weights)
    return out.reshape(B, D, 1)


# speedup vs baseline: 6.4600x; 1.6946x over previous
import functools

import jax
import jax.numpy as jnp
from jax import lax
from jax.experimental import pallas as pl
from jax.experimental.pallas import tpu as pltpu
from jax.experimental.pallas import tpu_sc as plsc

N_NODES = 10000
INPUTS_DIM = 32
D = 128
N_EDGES = N_NODES * INPUTS_DIM
B = N_EDGES // INPUTS_DIM

_INFO = plsc.get_sparse_core_info()
NC = _INFO.num_cores
NS = _INFO.num_subcores
NW = NC * NS
L = _INFO.num_lanes

CHUNK_ROWS = 8
CHUNK_IDX = CHUNK_ROWS * INPUTS_DIM
IDX_ROWS = CHUNK_IDX // 128
N_CHUNKS = B // CHUNK_ROWS
ITERS = -(-N_CHUNKS // NW)


def _tanh(x):
    e = jnp.exp(x * 2.0)
    return 1.0 - 2.0 / (e + 1.0)


def _body(table_hbm, ord_hbm, w_hbm, out_hbm,
          idx0, idx1, rows0, rows1, w_v, out0, out1,
          sem_i0, sem_i1, sem_g0, sem_g1, sem_o0, sem_o1):
    cid = lax.axis_index("c")
    sid = lax.axis_index("s")
    wid = sid * NC + cid

    idx = (idx0, idx1)
    rows = (rows0, rows1)
    outs = (out0, out1)
    sem_i = (sem_i0, sem_i1)
    sem_g = (sem_g0, sem_g1)
    sem_o = (sem_o0, sem_o1)

    pltpu.sync_copy(w_hbm, w_v)

    def chunk_id(t):
        return jnp.minimum(wid + t * NW, N_CHUNKS - 1)

    def ord_slice(c):
        return ord_hbm.at[pl.ds(c * IDX_ROWS, IDX_ROWS)]

    def issue_gathers(b):
        for j in range(IDX_ROWS):
            pltpu.async_copy(
                table_hbm.at[idx[b].at[j]],
                rows[b].at[pl.ds(j * 128, 128)],
                sem_g[b],
            )

    def wait_gathers(b):
        for j in range(IDX_ROWS):
            pltpu.make_async_copy(
                table_hbm.at[idx[b].at[j]],
                rows[b].at[pl.ds(j * 128, 128)],
                sem_g[b],
            ).wait()

    pltpu.sync_copy(ord_slice(chunk_id(0)), idx[0])
    issue_gathers(0)
    pltpu.async_copy(ord_slice(chunk_id(1)), idx[1], sem_i[1])

    def iter_body(p, carry):
        for b in range(2):
            t = p * 2 + b
            b1 = 1 - b
            wait_gathers(b)
            pltpu.async_copy(ord_slice(chunk_id(t + 2)), idx[b], sem_i[b])
            pltpu.make_async_copy(
                ord_slice(chunk_id(t + 1)), idx[b1], sem_i[b1]
            ).wait()
            issue_gathers(b1)
            @pl.when(t >= 2)
            def _():
                pltpu.make_async_copy(
                    outs[b],
                    out_hbm.at[pl.ds(chunk_id(t - 2) * CHUNK_ROWS, CHUNK_ROWS)],
                    sem_o[b],
                ).wait()
            c = chunk_id(t)

            def dbody(d8, rc):
                dsl = pl.ds(d8 * L, L)
                wvs = [w_v[i, dsl] for i in range(INPUTS_DIM)]
                for r in range(CHUNK_ROWS):
                    base = r * INPUTS_DIM
                    accs = [
                        wvs[i] * rows[b][base + i, dsl] for i in range(4)
                    ]
                    for i in range(4, INPUTS_DIM):
                        accs[i % 4] = accs[i % 4] + wvs[i] * rows[b][base + i, dsl]
                    acc = (accs[0] + accs[1]) + (accs[2] + accs[3])
                    outs[b][r, dsl] = _tanh(acc)
                return rc

            lax.fori_loop(0, D // L, dbody, 0)
            pltpu.async_copy(
                outs[b], out_hbm.at[pl.ds(c * CHUNK_ROWS, CHUNK_ROWS)], sem_o[b]
            )
        return carry

    lax.fori_loop(0, ITERS // 2, iter_body, 0)

    t_last = ITERS - 1
    pltpu.make_async_copy(ord_slice(chunk_id(t_last + 2)), idx[1], sem_i[1]).wait()
    wait_gathers(0)
    for b in range(2):
        t = t_last - 1 + b
        pltpu.make_async_copy(
            outs[b],
            out_hbm.at[pl.ds(chunk_id(t) * CHUNK_ROWS, CHUNK_ROWS)],
            sem_o[b],
        ).wait()


@jax.jit
def _run(table, ords2d, weights):
    mesh = plsc.VectorSubcoreMesh(core_axis_name="c", subcore_axis_name="s")
    f = functools.partial(
        pl.kernel,
        mesh=mesh,
        out_type=jax.ShapeDtypeStruct((B, D), jnp.float32),
        scratch_types=[
            pltpu.VMEM((IDX_ROWS, 128), jnp.int32),
            pltpu.VMEM((IDX_ROWS, 128), jnp.int32),
            pltpu.VMEM((CHUNK_IDX, D), jnp.float32),
            pltpu.VMEM((CHUNK_IDX, D), jnp.float32),
            pltpu.VMEM((INPUTS_DIM, D), jnp.float32),
            pltpu.VMEM((CHUNK_ROWS, D), jnp.float32),
            pltpu.VMEM((CHUNK_ROWS, D), jnp.float32),
            pltpu.SemaphoreType.DMA,
            pltpu.SemaphoreType.DMA,
            pltpu.SemaphoreType.DMA,
            pltpu.SemaphoreType.DMA,
            pltpu.SemaphoreType.DMA,
            pltpu.SemaphoreType.DMA,
        ],
    )(_body)
    return f(table, ords2d, weights)


def kernel(layer_values, ordinals, weights):
    table = layer_values.reshape(N_NODES, D)
    ords2d = ordinals.astype(jnp.int32).reshape(N_EDGES // 128, 128)
    out = _run(table, ords2d, weights)
    return out.reshape(B, D, 1)
